# baseline (speedup 1.0000x reference)
import functools

import jax
import jax.numpy as jnp
from jax import lax
from jax.experimental import pallas as pl
from jax.experimental.pallas import tpu as pltpu
from jax.experimental.pallas import tpu_sc as plsc

BATCH = 4096
SEQ = 200
EMBED = 64
LANES = 16
ED = EMBED // LANES

NC, NS = 2, 16
NW = NC * NS
BT = BATCH // NW
KP = 2
NBLK = SEQ // KP

CHUNK = 8 * BT
TRW = KP * EMBED * BT
OUT_LEN = SEQ * 8 * NW * CHUNK


def _pos_encoding(seq_len, d, n=10000.0):
    k = jnp.arange(seq_len, dtype=jnp.float32)[:, None]
    i = jnp.arange(d // 2, dtype=jnp.float32)[None, :]
    ang = k / jnp.power(n, 2.0 * i / d)
    p = jnp.zeros((seq_len, d), dtype=jnp.float32)
    p = p.at[:, 0::2].set(jnp.sin(ang))
    p = p.at[:, 1::2].set(jnp.cos(ang))
    return p


def _make_sc_kernel():
    mesh = plsc.VectorSubcoreMesh(core_axis_name="c", subcore_axis_name="s")

    @functools.partial(
        pl.kernel,
        mesh=mesh,
        compiler_params=pltpu.CompilerParams(
            use_tc_tiling_on_sc=False, needs_layout_passes=False),
        out_type=jax.ShapeDtypeStruct((OUT_LEN,), jnp.float32),
        scratch_types=[
            pltpu.VMEM((SEQ, EMBED), jnp.float32),
            pltpu.VMEM((KP, 1, BT), jnp.int32),
            pltpu.VMEM((KP, 1, BT), jnp.int32),
            pltpu.VMEM((KP, BT, EMBED), jnp.float32),
            pltpu.VMEM((KP, BT, EMBED), jnp.float32),
            pltpu.VMEM((TRW,), jnp.float32),
            pltpu.VMEM((TRW,), jnp.float32),
            pltpu.SemaphoreType.DMA,
            pltpu.SemaphoreType.DMA,
            pltpu.SemaphoreType.DMA,
            pltpu.SemaphoreType.DMA,
        ],
    )
    def emb_kernel(table_hbm, xt_hbm, p_hbm, out_hbm,
                   p_v, idx0, idx1, rows0, rows1, tr0, tr1,
                   semg0, semg1, semw0, semw1):
        wid = lax.axis_index("s") * NC + lax.axis_index("c")
        idx = (idx0, idx1)
        rows = (rows0, rows1)
        tr = (tr0, tr1)
        semg = (semg0, semg1)
        semw = (semw0, semw1)

        pltpu.sync_copy(p_hbm, p_v)

        iota = jax.lax.iota(jnp.int32, LANES)
        tbase = [[iota * BT + (kp * EMBED * BT + d * LANES * BT)
                  for d in range(ED)] for kp in range(KP)]

        def load_idx(buf, blk):
            pltpu.sync_copy(
                xt_hbm.at[pl.ds(blk * KP, KP), pl.ds(wid, 1)], idx[buf])

        def fire_gathers(buf):
            for kp in range(KP):
                pltpu.async_copy(
                    table_hbm.at[idx[buf].at[kp, 0]],
                    rows[buf].at[kp], semg[buf])

        def wait_gathers(buf):
            for kp in range(KP):
                pltpu.make_async_copy(
                    table_hbm.at[idx[buf].at[kp, 0]],
                    rows[buf].at[kp], semg[buf]).wait()

        def fire_wb(buf, blk):
            for kp in range(KP):
                for te in range(8):
                    src = tr[buf].at[pl.ds((kp * 8 + te) * CHUNK, CHUNK)]
                    off = (((blk * KP + kp) * 8 + te) * NW + wid) * CHUNK
                    pltpu.async_copy(src, out_hbm.at[pl.ds(off, CHUNK)],
                                     semw[buf])

        def wait_wb(buf):
            for _ in range(KP * 8):
                pltpu.make_async_copy(
                    tr[buf].at[pl.ds(0, CHUNK)],
                    out_hbm.at[pl.ds(0, CHUNK)], semw[buf]).wait()

        def do_block(buf, blk):
            p0 = blk * KP
            pes = [[p_v[p0 + kp, pl.ds(d * LANES, LANES)] for d in range(ED)]
                   for kp in range(KP)]

            def rbody(b, carry):
                bvec = jnp.broadcast_to(b, (LANES,))
                for kp in range(KP):
                    for d in range(ED):
                        val = rows[buf][kp, b, pl.ds(d * LANES, LANES)]
                        plsc.store_scatter(
                            tr[buf], [tbase[kp][d] + bvec],
                            val + pes[kp][d])
                return carry

            lax.fori_loop(0, BT, rbody, 0)

        def body(i, carry):
            blk_a = 2 * i
            blk_b = 2 * i + 1

            load_idx(1, blk_b)

            @pl.when(i > 0)
            def _():
                wait_wb(1)

            fire_gathers(1)

            wait_gathers(0)

            @pl.when(i > 0)
            def _():
                wait_wb(0)

            do_block(0, blk_a)
            fire_wb(0, blk_a)

            @pl.when(i < NBLK // 2 - 1)
            def _():
                load_idx(0, blk_a + 2)
                fire_gathers(0)

            wait_gathers(1)
            do_block(1, blk_b)
            fire_wb(1, blk_b)
            return carry

        load_idx(0, 0)
        fire_gathers(0)
        lax.fori_loop(0, NBLK // 2, body, 0)
        wait_wb(0)
        wait_wb(1)

    return emb_kernel


def kernel(table, x):
    p = _pos_encoding(SEQ, EMBED)
    xt = x.T.reshape(SEQ, NW, BT)
    out_phys = _make_sc_kernel()(table, xt, p)
    out = out_phys.reshape(SEQ, 8, NW, 8, BT)
    out = out.transpose(2, 4, 0, 1, 3).reshape(BATCH, SEQ, EMBED)
    return out


# speedup vs baseline: 1.3670x; 1.3670x over previous
import functools

import jax
import jax.numpy as jnp
from jax import lax
from jax.experimental import pallas as pl
from jax.experimental.pallas import tpu as pltpu
from jax.experimental.pallas import tpu_sc as plsc

BATCH = 4096
SEQ = 200
EMBED = 64
LANES = 16
ED = EMBED // LANES

NC, NS = 2, 16
NW = NC * NS
BT = BATCH // NW
KP = 2
NBLK = SEQ // KP

GDEPTH = 3
IDEPTH = 4


def _pos_encoding(seq_len, d, n=10000.0):
    k = jnp.arange(seq_len, dtype=jnp.float32)[:, None]
    i = jnp.arange(d // 2, dtype=jnp.float32)[None, :]
    ang = k / jnp.power(n, 2.0 * i / d)
    p = jnp.zeros((seq_len, d), dtype=jnp.float32)
    p = p.at[:, 0::2].set(jnp.sin(ang))
    p = p.at[:, 1::2].set(jnp.cos(ang))
    return p


def _make_sc_kernel():
    mesh = plsc.VectorSubcoreMesh(core_axis_name="c", subcore_axis_name="s")

    @functools.partial(
        pl.kernel,
        mesh=mesh,
        compiler_params=pltpu.CompilerParams(
            use_tc_tiling_on_sc=False, needs_layout_passes=False),
        out_type=jax.ShapeDtypeStruct((SEQ, 8, NW, 8, BT), jnp.float32),
        scratch_types=(
            [pltpu.VMEM((SEQ, EMBED), jnp.float32)]
            + [pltpu.VMEM((KP, 1, BT), jnp.int32) for _ in range(4)]
            + [pltpu.VMEM((KP, BT, EMBED), jnp.float32) for _ in range(4)]
            + [pltpu.VMEM((KP, 8, 1, 8, BT), jnp.float32) for _ in range(2)]
            + [pltpu.SemaphoreType.DMA for _ in range(10)]
        ),
    )
    def emb_kernel(table_hbm, xt_hbm, p_hbm, out_hbm,
                   p_v, idx0, idx1, idx2, idx3,
                   rows0, rows1, rows2, rows3, tr0, tr1,
                   semi0, semi1, semi2, semi3,
                   semg0, semg1, semg2, semg3, semw0, semw1):
        wid = lax.axis_index("s") * NC + lax.axis_index("c")
        idx = (idx0, idx1, idx2, idx3)
        rows = (rows0, rows1, rows2, rows3)
        tr = (tr0, tr1)
        semi = (semi0, semi1, semi2, semi3)
        semg = (semg0, semg1, semg2, semg3)
        semw = (semw0, semw1)

        pltpu.sync_copy(p_hbm, p_v)

        iota = jax.lax.iota(jnp.int32, LANES)
        io8 = iota >> 3
        el = iota & 7
        zero16 = iota & 0
        kpvec = [iota * 0 + kp for kp in range(KP)]
        tevec = [io8 + 2 * d for d in range(ED)]

        def idx_src(blk):
            return xt_hbm.at[pl.ds(blk * KP, KP), pl.ds(wid, 1)]

        def fire_idx(buf, blk):
            pltpu.async_copy(idx_src(blk), idx[buf], semi[buf])

        def wait_idx(buf):
            pltpu.make_async_copy(idx_src(0), idx[buf], semi[buf]).wait()

        def fire_gathers(buf):
            for kp in range(KP):
                pltpu.async_copy(
                    table_hbm.at[idx[buf].at[kp, 0]],
                    rows[buf].at[kp], semg[buf])

        def wait_gathers(buf):
            for kp in range(KP):
                pltpu.make_async_copy(
                    table_hbm.at[idx[buf].at[kp, 0]],
                    rows[buf].at[kp], semg[buf]).wait()

        def out_slice(blk):
            return out_hbm.at[pl.ds(blk * KP, KP), :, pl.ds(wid, 1)]

        def fire_wb(t, blk):
            pltpu.async_copy(tr[t], out_slice(blk), semw[t])

        def wait_wb(t):
            pltpu.make_async_copy(tr[t], out_slice(0), semw[t]).wait()

        def do_block(buf, t, blk):
            p0 = blk * KP
            pes = [[p_v[p0 + kp, pl.ds(d * LANES, LANES)] for d in range(ED)]
                   for kp in range(KP)]

            @plsc.parallel_loop(0, BT, unroll=4, carry=jnp.int32(0))
            def rbody(b, carry):
                bvec = jnp.broadcast_to(b, (LANES,))
                for kp in range(KP):
                    for d in range(ED):
                        val = rows[buf][kp, b, pl.ds(d * LANES, LANES)]
                        plsc.store_scatter(
                            tr[t],
                            [kpvec[kp], tevec[d], zero16, el, bvec],
                            val + pes[kp][d])
                return carry

        for blk in range(IDEPTH):
            fire_idx(blk, blk)
        for blk in range(GDEPTH):
            wait_idx(blk)
            fire_gathers(blk)

        def body(i, carry):
            for j in range(4):
                blk = 4 * i + j

                wait_gathers(j)

                nxt_i = blk + IDEPTH

                @pl.when(nxt_i < NBLK)
                def _(nxt_i=nxt_i, j=j):
                    fire_idx(j, nxt_i)

                nxt_g = blk + GDEPTH
                gn = (j + GDEPTH) & 3

                @pl.when(nxt_g < NBLK)
                def _(gn=gn):
                    wait_idx(gn)
                    fire_gathers(gn)
                if j < 2:
                    @pl.when(blk >= 2)
                    def _(j=j):
                        wait_wb(j & 1)
                else:
                    wait_wb(j & 1)
                do_block(j, j & 1, blk)
                fire_wb(j & 1, blk)
            return carry

        lax.fori_loop(0, NBLK // 4, body, 0)
        wait_wb(0)
        wait_wb(1)

    return emb_kernel


def kernel(table, x):
    p = _pos_encoding(SEQ, EMBED)
    xt = x.T.reshape(SEQ, NW, BT)
    out_phys = _make_sc_kernel()(table, xt, p)
    out = out_phys.transpose(2, 4, 0, 1, 3).reshape(BATCH, SEQ, EMBED)
    return out


# speedup vs baseline: 2.3613x; 1.7273x over previous
import functools

import jax
import jax.numpy as jnp
from jax import lax
from jax.experimental import pallas as pl
from jax.experimental.pallas import tpu as pltpu
from jax.experimental.pallas import tpu_sc as plsc

BATCH = 4096
SEQ = 200
EMBED = 64
LANES = 16
ED = EMBED // LANES

NC, NS = 2, 16
NW = NC * NS
BT = BATCH // NW
KP = 2
NBLK = SEQ // KP

GDEPTH = 3
IDEPTH = 4


def _pos_encoding(seq_len, d, n=10000.0):
    k = jnp.arange(seq_len, dtype=jnp.float32)[:, None]
    i = jnp.arange(d // 2, dtype=jnp.float32)[None, :]
    ang = k / jnp.power(n, 2.0 * i / d)
    p = jnp.zeros((seq_len, d), dtype=jnp.float32)
    p = p.at[:, 0::2].set(jnp.sin(ang))
    p = p.at[:, 1::2].set(jnp.cos(ang))
    return p


def _make_sc_kernel():
    mesh = plsc.VectorSubcoreMesh(core_axis_name="c", subcore_axis_name="s")

    @functools.partial(
        pl.kernel,
        mesh=mesh,
        compiler_params=pltpu.CompilerParams(
            use_tc_tiling_on_sc=False, needs_layout_passes=False),
        out_type=jax.ShapeDtypeStruct((SEQ * 8 * NW * 8, BT), jnp.float32),
        scratch_types=(
            [pltpu.VMEM((SEQ, EMBED), jnp.float32)]
            + [pltpu.VMEM((KP, 1, BT), jnp.int32) for _ in range(4)]
            + [pltpu.VMEM((KP, BT, EMBED), jnp.float32) for _ in range(4)]
            + [pltpu.VMEM((KP * EMBED, BT + 1), jnp.float32)
               for _ in range(2)]
            + [pltpu.SemaphoreType.DMA for _ in range(10)]
        ),
    )
    def emb_kernel(table_hbm, xt_hbm, p_hbm, out_hbm,
                   p_v, idx0, idx1, idx2, idx3,
                   rows0, rows1, rows2, rows3, tr0, tr1,
                   semi0, semi1, semi2, semi3,
                   semg0, semg1, semg2, semg3, semw0, semw1):
        wid = lax.axis_index("s") * NC + lax.axis_index("c")
        idx = (idx0, idx1, idx2, idx3)
        rows = (rows0, rows1, rows2, rows3)
        tr = (tr0, tr1)
        semi = (semi0, semi1, semi2, semi3)
        semg = (semg0, semg1, semg2, semg3)
        semw = (semw0, semw1)

        pltpu.sync_copy(p_hbm, p_v)

        iota = jax.lax.iota(jnp.int32, LANES)
        trow = [[iota + (kp * EMBED + d * LANES) for d in range(ED)]
                for kp in range(KP)]

        def idx_src(blk):
            return xt_hbm.at[pl.ds(blk * KP, KP), pl.ds(wid, 1)]

        def fire_idx(buf, blk):
            pltpu.async_copy(idx_src(blk), idx[buf], semi[buf])

        def wait_idx(buf):
            pltpu.make_async_copy(idx_src(0), idx[buf], semi[buf]).wait()

        def fire_gathers(buf):
            for kp in range(KP):
                pltpu.async_copy(
                    table_hbm.at[idx[buf].at[kp, 0]],
                    rows[buf].at[kp], semg[buf])

        def wait_gathers(buf):
            for kp in range(KP):
                pltpu.make_async_copy(
                    table_hbm.at[idx[buf].at[kp, 0]],
                    rows[buf].at[kp], semg[buf]).wait()

        def fire_wb(t, blk):
            for kp in range(KP):
                for te in range(8):
                    src = tr[t].at[pl.ds((kp * 8 + te) * 8, 8), pl.ds(0, BT)]
                    r0 = (((blk * KP + kp) * 8 + te) * NW + wid) * 8
                    pltpu.async_copy(src, out_hbm.at[pl.ds(r0, 8)], semw[t])

        def wait_wb(t):
            for _ in range(KP * 8):
                pltpu.make_async_copy(
                    tr[t].at[pl.ds(0, 8), pl.ds(0, BT)],
                    out_hbm.at[pl.ds(0, 8)], semw[t]).wait()

        def do_block(buf, t, blk):
            p0 = blk * KP
            pes = [[p_v[p0 + kp, pl.ds(d * LANES, LANES)] for d in range(ED)]
                   for kp in range(KP)]

            @plsc.parallel_loop(0, BT, unroll=4, carry=jnp.int32(0))
            def rbody(b, carry):
                bvec = jnp.broadcast_to(b, (LANES,))
                for kp in range(KP):
                    for d in range(ED):
                        val = rows[buf][kp, b, pl.ds(d * LANES, LANES)]
                        plsc.store_scatter(
                            tr[t], [trow[kp][d], bvec],
                            val + pes[kp][d])
                return carry

        for blk in range(IDEPTH):
            fire_idx(blk, blk)
        for blk in range(GDEPTH):
            wait_idx(blk)
            fire_gathers(blk)

        def body(i, carry):
            for j in range(4):
                blk = 4 * i + j

                wait_gathers(j)

                nxt_i = blk + IDEPTH

                @pl.when(nxt_i < NBLK)
                def _(nxt_i=nxt_i, j=j):
                    fire_idx(j, nxt_i)

                nxt_g = blk + GDEPTH
                gn = (j + GDEPTH) & 3

                @pl.when(nxt_g < NBLK)
                def _(gn=gn):
                    wait_idx(gn)
                    fire_gathers(gn)
                if j < 2:
                    @pl.when(blk >= 2)
                    def _(j=j):
                        wait_wb(j & 1)
                else:
                    wait_wb(j & 1)
                do_block(j, j & 1, blk)
                fire_wb(j & 1, blk)
            return carry

        lax.fori_loop(0, NBLK // 4, body, 0)
        wait_wb(0)
        wait_wb(1)

    return emb_kernel


def kernel(table, x):
    p = _pos_encoding(SEQ, EMBED)
    xt = x.T.reshape(SEQ, NW, BT)
    out_phys = _make_sc_kernel()(table, xt, p)
    out = out_phys.reshape(SEQ, 8, NW, 8, BT)
    out = out.transpose(2, 4, 0, 1, 3).reshape(BATCH, SEQ, EMBED)
    return out


# speedup vs baseline: 2.3624x; 1.0005x over previous
import functools

import jax
import jax.numpy as jnp
from jax import lax
from jax.experimental import pallas as pl
from jax.experimental.pallas import tpu as pltpu
from jax.experimental.pallas import tpu_sc as plsc

BATCH = 4096
SEQ = 200
EMBED = 64
LANES = 16
ED = EMBED // LANES

NC, NS = 2, 16
NW = NC * NS
BT = BATCH // NW
KP = 2
NBLK = SEQ // KP

GDEPTH = 3
IDEPTH = 4


def _pos_encoding(seq_len, d, n=10000.0):
    k = jnp.arange(seq_len, dtype=jnp.float32)[:, None]
    i = jnp.arange(d // 2, dtype=jnp.float32)[None, :]
    ang = k / jnp.power(n, 2.0 * i / d)
    p = jnp.zeros((seq_len, d), dtype=jnp.float32)
    p = p.at[:, 0::2].set(jnp.sin(ang))
    p = p.at[:, 1::2].set(jnp.cos(ang))
    return p


def _make_sc_kernel():
    mesh = plsc.VectorSubcoreMesh(core_axis_name="c", subcore_axis_name="s")

    @functools.partial(
        pl.kernel,
        mesh=mesh,
        compiler_params=pltpu.CompilerParams(
            use_tc_tiling_on_sc=False, needs_layout_passes=False),
        out_type=jax.ShapeDtypeStruct((SEQ * 8 * NW * 8, BT), jnp.float32),
        scratch_types=(
            [pltpu.VMEM((SEQ, EMBED), jnp.float32)]
            + [pltpu.VMEM((1, 1, KP * BT), jnp.int32) for _ in range(4)]
            + [pltpu.VMEM((KP * BT, EMBED), jnp.float32) for _ in range(4)]
            + [pltpu.VMEM((KP * EMBED, BT + 1), jnp.float32)
               for _ in range(2)]
            + [pltpu.SemaphoreType.DMA for _ in range(10)]
        ),
    )
    def emb_kernel(table_hbm, xt_hbm, p_hbm, out_hbm,
                   p_v, idx0, idx1, idx2, idx3,
                   rows0, rows1, rows2, rows3, tr0, tr1,
                   semi0, semi1, semi2, semi3,
                   semg0, semg1, semg2, semg3, semw0, semw1):
        wid = lax.axis_index("s") * NC + lax.axis_index("c")
        idx = (idx0, idx1, idx2, idx3)
        rows = (rows0, rows1, rows2, rows3)
        tr = (tr0, tr1)
        semi = (semi0, semi1, semi2, semi3)
        semg = (semg0, semg1, semg2, semg3)
        semw = (semw0, semw1)

        pltpu.sync_copy(p_hbm, p_v)

        iota = jax.lax.iota(jnp.int32, LANES)
        trow = [[iota + (kp * EMBED + d * LANES) for d in range(ED)]
                for kp in range(KP)]

        def idx_src(blk):
            return xt_hbm.at[pl.ds(blk, 1), pl.ds(wid, 1)]

        def fire_idx(buf, blk):
            pltpu.async_copy(idx_src(blk), idx[buf], semi[buf])

        def wait_idx(buf):
            pltpu.make_async_copy(idx_src(0), idx[buf], semi[buf]).wait()

        def fire_gathers(buf):
            pltpu.async_copy(
                table_hbm.at[idx[buf].at[0, 0]], rows[buf], semg[buf])

        def wait_gathers(buf):
            pltpu.make_async_copy(
                table_hbm.at[idx[buf].at[0, 0]], rows[buf], semg[buf]).wait()

        def fire_wb(t, blk):
            for kp in range(KP):
                for te in range(8):
                    src = tr[t].at[pl.ds((kp * 8 + te) * 8, 8), pl.ds(0, BT)]
                    r0 = (((blk * KP + kp) * 8 + te) * NW + wid) * 8
                    pltpu.async_copy(src, out_hbm.at[pl.ds(r0, 8)], semw[t])

        def wait_wb(t):
            for _ in range(KP * 8):
                pltpu.make_async_copy(
                    tr[t].at[pl.ds(0, 8), pl.ds(0, BT)],
                    out_hbm.at[pl.ds(0, 8)], semw[t]).wait()

        def do_block(buf, t, blk):
            p0 = blk * KP
            pes = [[p_v[p0 + kp, pl.ds(d * LANES, LANES)] for d in range(ED)]
                   for kp in range(KP)]

            @plsc.parallel_loop(0, BT, unroll=4, carry=jnp.int32(0))
            def rbody(b, carry):
                bvec = jnp.broadcast_to(b, (LANES,))
                for kp in range(KP):
                    for d in range(ED):
                        val = rows[buf][kp * BT + b, pl.ds(d * LANES, LANES)]
                        plsc.store_scatter(
                            tr[t], [trow[kp][d], bvec],
                            val + pes[kp][d])
                return carry

        for blk in range(IDEPTH):
            fire_idx(blk, blk)
        for blk in range(GDEPTH):
            wait_idx(blk)
            fire_gathers(blk)

        def body(i, carry):
            for j in range(4):
                blk = 4 * i + j

                wait_gathers(j)

                nxt_i = blk + IDEPTH

                @pl.when(nxt_i < NBLK)
                def _(nxt_i=nxt_i, j=j):
                    fire_idx(j, nxt_i)

                nxt_g = blk + GDEPTH
                gn = (j + GDEPTH) & 3

                @pl.when(nxt_g < NBLK)
                def _(gn=gn):
                    wait_idx(gn)
                    fire_gathers(gn)
                if j < 2:
                    @pl.when(blk >= 2)
                    def _(j=j):
                        wait_wb(j & 1)
                else:
                    wait_wb(j & 1)
                do_block(j, j & 1, blk)
                fire_wb(j & 1, blk)
            return carry

        lax.fori_loop(0, NBLK // 4, body, 0)
        wait_wb(0)
        wait_wb(1)

    return emb_kernel


def kernel(table, x):
    p = _pos_encoding(SEQ, EMBED)
    xt = x.T.reshape(NBLK, KP, NW, BT).transpose(0, 2, 1, 3)
    xt = xt.reshape(NBLK, NW, KP * BT)
    out_phys = _make_sc_kernel()(table, xt, p)
    out = out_phys.reshape(SEQ, 8, NW, 8, BT)
    out = out.transpose(2, 4, 0, 1, 3).reshape(BATCH, SEQ, EMBED)
    return out
